# baseline (speedup 1.0000x reference)
import functools

import jax
import jax.numpy as jnp
from jax import lax
from jax.experimental import pallas as pl
from jax.experimental.pallas import tpu as pltpu

N = 10000
E = 160000
F_IN = 128
F_E = 16
D = 32
B = 64
STEPS = 3
JUMPS = 3

TILE_E = 2000


def _prep_body(x_ref, w_ref, b_ref, o_ref):
    o_ref[...] = jax.nn.relu(
        jnp.dot(x_ref[...], w_ref[...], preferred_element_type=jnp.float32, precision=lax.Precision.HIGHEST)
        + b_ref[...]
    )


def _prep(x, W_lin0, b_lin0):
    return pl.pallas_call(
        _prep_body,
        out_shape=jax.ShapeDtypeStruct((N, D), jnp.float32),
    )(x, W_lin0, b_lin0.reshape(1, D))


def _msg_body(ea_ref, xj_ref, we1_ref, be1_ref, we2_ref, be2_ref, o_ref):
    u = jax.nn.relu(
        jnp.dot(ea_ref[...], we1_ref[...], preferred_element_type=jnp.float32, precision=lax.Precision.HIGHEST)
        + be1_ref[...]
    )
    w = (
        jnp.dot(u, we2_ref[...], preferred_element_type=jnp.float32, precision=lax.Precision.HIGHEST)
        + be2_ref[...]
    )
    w3 = w.reshape(TILE_E, D, D)
    o_ref[...] = jnp.sum(w3 * xj_ref[...][:, :, None], axis=1)


def _msg(edge_attr, x_j, W_e1, b_e1, W_e2, b_e2):
    grid = (E // TILE_E,)
    return pl.pallas_call(
        _msg_body,
        grid=grid,
        in_specs=[
            pl.BlockSpec((TILE_E, F_E), lambda i: (i, 0)),
            pl.BlockSpec((TILE_E, D), lambda i: (i, 0)),
            pl.BlockSpec((F_E, 128), lambda i: (0, 0)),
            pl.BlockSpec((1, 128), lambda i: (0, 0)),
            pl.BlockSpec((128, D * D), lambda i: (0, 0)),
            pl.BlockSpec((1, D * D), lambda i: (0, 0)),
        ],
        out_specs=pl.BlockSpec((TILE_E, D), lambda i: (i, 0)),
        out_shape=jax.ShapeDtypeStruct((E, D), jnp.float32),
    )(edge_attr, x_j, W_e1, b_e1.reshape(1, 128), W_e2, b_e2.reshape(1, D * D))


def _gru_body(agg_ref, deg_ref, out_ref, h_ref, wroot_ref, bconv_ref,
              wih_ref, whh_ref, bih_ref, bhh_ref, o_ref):
    agg = agg_ref[...] / deg_ref[...]
    m = jax.nn.relu(
        agg
        + jnp.dot(out_ref[...], wroot_ref[...], preferred_element_type=jnp.float32, precision=lax.Precision.HIGHEST)
        + bconv_ref[...]
    )
    gi = jnp.dot(m, wih_ref[...], preferred_element_type=jnp.float32, precision=lax.Precision.HIGHEST) + bih_ref[...]
    gh = jnp.dot(h_ref[...], whh_ref[...], preferred_element_type=jnp.float32, precision=lax.Precision.HIGHEST) + bhh_ref[...]
    r = jax.nn.sigmoid(gi[:, :D] + gh[:, :D])
    z = jax.nn.sigmoid(gi[:, D:2 * D] + gh[:, D:2 * D])
    n = jnp.tanh(gi[:, 2 * D:] + r * gh[:, 2 * D:])
    o_ref[...] = (1.0 - z) * n + z * h_ref[...]


def _gru(agg, deg_col, out, h, W_root, b_conv, W_ihT, W_hhT, b_ih, b_hh):
    return pl.pallas_call(
        _gru_body,
        out_shape=jax.ShapeDtypeStruct((N, D), jnp.float32),
    )(agg, deg_col, out, h, W_root, b_conv.reshape(1, D),
      W_ihT, W_hhT, b_ih.reshape(1, 3 * D), b_hh.reshape(1, 3 * D))


def _s2s_body(out_ref, batch_ref, wli_ref, wlh_ref, bli_ref, blh_ref,
              wlin1_ref, blin1_ref, wlin2_ref, blin2_ref, o_ref):
    out = out_ref[...]
    seg = batch_ref[...]
    cols = lax.broadcasted_iota(jnp.int32, (N, B), 1)
    S = (seg == cols).astype(jnp.float32)

    q_star = jnp.zeros((B, 2 * D), jnp.float32)
    hl = jnp.zeros((B, D), jnp.float32)
    cl = jnp.zeros((B, D), jnp.float32)
    for _ in range(STEPS):
        g = (
            jnp.dot(q_star, wli_ref[...], preferred_element_type=jnp.float32, precision=lax.Precision.HIGHEST)
            + bli_ref[...]
            + jnp.dot(hl, wlh_ref[...], preferred_element_type=jnp.float32, precision=lax.Precision.HIGHEST)
            + blh_ref[...]
        )
        ig = jax.nn.sigmoid(g[:, :D])
        fg = jax.nn.sigmoid(g[:, D:2 * D])
        cg = jnp.tanh(g[:, 2 * D:3 * D])
        og = jax.nn.sigmoid(g[:, 3 * D:])
        cl = fg * cl + ig * cg
        hl = og * jnp.tanh(cl)
        q = hl
        qn = jnp.dot(S, q, preferred_element_type=jnp.float32, precision=lax.Precision.HIGHEST)
        e = jnp.sum(out * qn, axis=1, keepdims=True)
        emax = jnp.max(jnp.where(S > 0, e, -jnp.inf), axis=0, keepdims=True)
        emax_n = jnp.sum(S * emax, axis=1, keepdims=True)
        ee = jnp.exp(e - emax_n)
        denom = lax.dot_general(
            S, ee, (((0,), (0,)), ((), ())),
            preferred_element_type=jnp.float32, precision=lax.Precision.HIGHEST)
        denom_n = jnp.dot(S, denom, preferred_element_type=jnp.float32, precision=lax.Precision.HIGHEST)
        a = ee / (denom_n + 1e-16)
        rvec = lax.dot_general(
            S, a * out, (((0,), (0,)), ((), ())),
            preferred_element_type=jnp.float32, precision=lax.Precision.HIGHEST)
        q_star = jnp.concatenate([q, rvec], axis=1)
    res = (
        jnp.dot(
            jax.nn.relu(
                jnp.dot(q_star, wlin1_ref[...], preferred_element_type=jnp.float32, precision=lax.Precision.HIGHEST)
                + blin1_ref[...]
            ),
            wlin2_ref[...],
            preferred_element_type=jnp.float32,
        )
        + blin2_ref[...]
    )
    o_ref[...] = res


def _s2s(out, batch, W_li, W_lh, b_li, b_lh, W_lin1, b_lin1, W_lin2, b_lin2):
    return pl.pallas_call(
        _s2s_body,
        out_shape=jax.ShapeDtypeStruct((B, 1), jnp.float32),
    )(out, batch.reshape(N, 1), W_li.T, W_lh.T,
      b_li.reshape(1, 4 * D), b_lh.reshape(1, 4 * D),
      W_lin1, b_lin1.reshape(1, D), W_lin2, b_lin2.reshape(1, 1))


def kernel(x, edge_index, edge_attr, batch, W_lin0, b_lin0, W_e1, b_e1,
           W_e2, b_e2, W_root, b_conv, W_ih, W_hh, b_ih, b_hh,
           W_li, W_lh, b_li, b_lh, W_lin1, b_lin1, W_lin2, b_lin2):
    src = edge_index[0]
    dst = edge_index[1]

    out = _prep(x, W_lin0, b_lin0)
    h = out

    deg = jax.ops.segment_sum(jnp.ones((E,), jnp.float32), dst, num_segments=N)
    deg_col = jnp.maximum(deg, 1.0).reshape(N, 1)

    W_ihT = W_ih.T
    W_hhT = W_hh.T

    for _ in range(JUMPS):
        x_j = jnp.take(out, src, axis=0)
        msg = _msg(edge_attr, x_j, W_e1, b_e1, W_e2, b_e2)
        agg = jax.ops.segment_sum(msg, dst, num_segments=N)
        h = _gru(agg, deg_col, out, h, W_root, b_conv, W_ihT, W_hhT, b_ih, b_hh)
        out = h

    res = _s2s(out, batch, W_li, W_lh, b_li, b_lh, W_lin1, b_lin1, W_lin2, b_lin2)
    return res.reshape(-1)


# speedup vs baseline: 1.1482x; 1.1482x over previous
import functools

import jax
import jax.numpy as jnp
from jax import lax
from jax.experimental import pallas as pl
from jax.experimental.pallas import tpu as pltpu
from jax.experimental.pallas import tpu_sc as plsc

N = 10000
E = 160000
F_IN = 128
F_E = 16
D = 32
B = 64
STEPS = 3
JUMPS = 3

TILE_E = 2000

NC = 2
NS = 16
NW = NC * NS
CHUNK = 200
N_CH = E // CHUNK // NW
N_PAD = 10240
ROWS_PER_TILE = N_PAD // NS

@functools.cache
def _sc_mesh():
    return plsc.VectorSubcoreMesh(
        core_axis_name="c", subcore_axis_name="s",
        num_cores=NC, num_subcores=NS)


CHUNK_G = 200
N_CH_G = E // CHUNK_G // NW


def _sc_gather(table, idx):
    @functools.partial(
        pl.kernel,
        mesh=_sc_mesh(),
        out_type=jax.ShapeDtypeStruct((E, 128), jnp.float32),
        scratch_types=[
            pltpu.VMEM((CHUNK_G,), jnp.int32),
            pltpu.VMEM((CHUNK_G, 128), jnp.float32),
            pltpu.SemaphoreType.DMA,
        ],
    )
    def k(table_hbm, idx_hbm, out_hbm, idx_v, rows_v, sem):
        wid = lax.axis_index("s") * NC + lax.axis_index("c")

        def body(i, carry):
            off = pl.multiple_of((i * NW + wid) * CHUNK_G, 8)
            pltpu.sync_copy(idx_hbm.at[pl.ds(off, CHUNK_G)], idx_v)
            pltpu.async_copy(table_hbm.at[idx_v], rows_v, sem).wait()
            pltpu.sync_copy(rows_v, out_hbm.at[pl.ds(off, CHUNK_G)])
            return carry

        lax.fori_loop(0, N_CH_G, body, 0)

    return k(table, idx)


def _sc_scatter(msgs, dst, zeros_nd):
    @functools.partial(
        pl.kernel,
        mesh=_sc_mesh(),
        out_type=jax.ShapeDtypeStruct((NC, N_PAD, D), jnp.float32),
        scratch_types=[
            pltpu.VMEM((CHUNK,), jnp.int32),
            pltpu.VMEM((CHUNK, D), jnp.float32),
            pltpu.VMEM_SHARED((N_PAD, D), jnp.float32),
        ],
    )
    def k(msg_hbm, dst_hbm, zero_hbm, out_hbm, idx_v, rows_v, acc):
        cid = lax.axis_index("c")
        sid = lax.axis_index("s")
        stripe = pl.multiple_of(sid * ROWS_PER_TILE, 8)
        pltpu.sync_copy(zero_hbm.at[pl.ds(stripe, ROWS_PER_TILE)],
                        acc.at[pl.ds(stripe, ROWS_PER_TILE)])
        plsc.subcore_barrier()
        wid = sid * NC + cid

        def body(i, carry):
            off = pl.multiple_of((i * NW + wid) * CHUNK, 8)
            pltpu.sync_copy(dst_hbm.at[pl.ds(off, CHUNK)], idx_v)
            pltpu.sync_copy(msg_hbm.at[pl.ds(off, CHUNK)], rows_v)
            pltpu.sync_copy(rows_v, acc.at[idx_v], add=True)
            return carry

        lax.fori_loop(0, N_CH, body, 0)
        plsc.subcore_barrier()
        pltpu.sync_copy(acc.at[pl.ds(stripe, ROWS_PER_TILE)],
                        out_hbm.at[cid].at[pl.ds(stripe, ROWS_PER_TILE)])

    return k(msgs, dst, zeros_nd)


DEG_CHUNK = 200
N_CH_DEG = E // DEG_CHUNK // NW


def _sc_deg(dst, ones_chunk, zeros_n16):
    @functools.partial(
        pl.kernel,
        mesh=_sc_mesh(),
        out_type=jax.ShapeDtypeStruct((NC, N_PAD, 16), jnp.float32),
        scratch_types=[
            pltpu.VMEM((DEG_CHUNK,), jnp.int32),
            pltpu.VMEM((DEG_CHUNK, 16), jnp.float32),
            pltpu.VMEM_SHARED((N_PAD, 16), jnp.float32),
        ],
    )
    def k(dst_hbm, ones_hbm, zero_hbm, out_hbm, idx_v, rows_v, acc):
        cid = lax.axis_index("c")
        sid = lax.axis_index("s")
        stripe = pl.multiple_of(sid * ROWS_PER_TILE, 8)
        pltpu.sync_copy(zero_hbm.at[pl.ds(stripe, ROWS_PER_TILE)],
                        acc.at[pl.ds(stripe, ROWS_PER_TILE)])
        pltpu.sync_copy(ones_hbm, rows_v)
        plsc.subcore_barrier()
        wid = sid * NC + cid

        def body(i, carry):
            off = pl.multiple_of((i * NW + wid) * DEG_CHUNK, 8)
            pltpu.sync_copy(dst_hbm.at[pl.ds(off, DEG_CHUNK)], idx_v)
            pltpu.sync_copy(rows_v, acc.at[idx_v], add=True)
            return carry

        lax.fori_loop(0, N_CH_DEG, body, 0)
        plsc.subcore_barrier()
        pltpu.sync_copy(acc.at[pl.ds(stripe, ROWS_PER_TILE)],
                        out_hbm.at[cid].at[pl.ds(stripe, ROWS_PER_TILE)])

    return k(dst, ones_chunk, zeros_n16)


def _prep_body(x_ref, w_ref, b_ref, o_ref):
    h0 = jax.nn.relu(
        jnp.dot(x_ref[...], w_ref[...], preferred_element_type=jnp.float32)
        + b_ref[...]
    )
    o_ref[...] = jnp.pad(h0, ((0, 0), (0, 128 - D)))


TILE_N = 1024


def _prep(x, W_lin0, b_lin0):
    x_pad = jnp.pad(x, ((0, N_PAD - N), (0, 0)))
    return pl.pallas_call(
        _prep_body,
        grid=(N_PAD // TILE_N,),
        in_specs=[
            pl.BlockSpec((TILE_N, F_IN), lambda i: (i, 0)),
            pl.BlockSpec((F_IN, D), lambda i: (0, 0)),
            pl.BlockSpec((1, D), lambda i: (0, 0)),
        ],
        out_specs=pl.BlockSpec((TILE_N, 128), lambda i: (i, 0)),
        out_shape=jax.ShapeDtypeStruct((N_PAD, 128), jnp.float32),
    )(x_pad, W_lin0, b_lin0.reshape(1, D))


def _msg_body(ea_ref, xj_ref, we1_ref, be1_ref, we2_ref, be2_ref, o_ref):
    u = jax.nn.relu(
        jnp.dot(ea_ref[...], we1_ref[...], preferred_element_type=jnp.float32)
        + be1_ref[...]
    )
    w = (
        jnp.dot(u, we2_ref[...], preferred_element_type=jnp.float32)
        + be2_ref[...]
    )
    w3 = w.reshape(TILE_E, D, D)
    xj = xj_ref[:, :D]
    o_ref[...] = jnp.sum(w3 * xj[:, :, None], axis=1)


def _msg(edge_attr, x_j, W_e1, b_e1, W_e2, b_e2):
    grid = (E // TILE_E,)
    return pl.pallas_call(
        _msg_body,
        grid=grid,
        in_specs=[
            pl.BlockSpec((TILE_E, F_E), lambda i: (i, 0)),
            pl.BlockSpec((TILE_E, 128), lambda i: (i, 0)),
            pl.BlockSpec((F_E, 128), lambda i: (0, 0)),
            pl.BlockSpec((1, 128), lambda i: (0, 0)),
            pl.BlockSpec((128, D * D), lambda i: (0, 0)),
            pl.BlockSpec((1, D * D), lambda i: (0, 0)),
        ],
        out_specs=pl.BlockSpec((TILE_E, D), lambda i: (i, 0)),
        out_shape=jax.ShapeDtypeStruct((E, D), jnp.float32),
    )(edge_attr, x_j, W_e1, b_e1.reshape(1, 128), W_e2, b_e2.reshape(1, D * D))


def _gru_body(p_ref, degp_ref, out_ref, wroot_ref, bconv_ref,
              wih_ref, whh_ref, bih_ref, bhh_ref, o_ref):
    deg = jnp.maximum(degp_ref[0, :, 0:1] + degp_ref[1, :, 0:1], 1.0)
    agg = (p_ref[0] + p_ref[1]) / deg
    h_prev = out_ref[:, :D]
    m = jax.nn.relu(
        agg
        + jnp.dot(h_prev, wroot_ref[...], preferred_element_type=jnp.float32)
        + bconv_ref[...]
    )
    gi = jnp.dot(m, wih_ref[...], preferred_element_type=jnp.float32) + bih_ref[...]
    gh = jnp.dot(h_prev, whh_ref[...], preferred_element_type=jnp.float32) + bhh_ref[...]
    r = jax.nn.sigmoid(gi[:, :D] + gh[:, :D])
    z = jax.nn.sigmoid(gi[:, D:2 * D] + gh[:, D:2 * D])
    n = jnp.tanh(gi[:, 2 * D:] + r * gh[:, 2 * D:])
    h_new = (1.0 - z) * n + z * h_prev
    o_ref[...] = jnp.pad(h_new, ((0, 0), (0, 128 - D)))


def _gru(p, degp, out, W_root, b_conv, W_ihT, W_hhT, b_ih, b_hh):
    return pl.pallas_call(
        _gru_body,
        grid=(N_PAD // TILE_N,),
        in_specs=[
            pl.BlockSpec((NC, TILE_N, D), lambda i: (0, i, 0)),
            pl.BlockSpec((NC, TILE_N, 16), lambda i: (0, i, 0)),
            pl.BlockSpec((TILE_N, 128), lambda i: (i, 0)),
            pl.BlockSpec((D, D), lambda i: (0, 0)),
            pl.BlockSpec((1, D), lambda i: (0, 0)),
            pl.BlockSpec((D, 3 * D), lambda i: (0, 0)),
            pl.BlockSpec((D, 3 * D), lambda i: (0, 0)),
            pl.BlockSpec((1, 3 * D), lambda i: (0, 0)),
            pl.BlockSpec((1, 3 * D), lambda i: (0, 0)),
        ],
        out_specs=pl.BlockSpec((TILE_N, 128), lambda i: (i, 0)),
        out_shape=jax.ShapeDtypeStruct((N_PAD, 128), jnp.float32),
    )(p, degp, out, W_root, b_conv.reshape(1, D),
      W_ihT, W_hhT, b_ih.reshape(1, 3 * D), b_hh.reshape(1, 3 * D))


def _s2s_body(out_ref, batch_ref, wli_ref, wlh_ref, bli_ref, blh_ref,
              wlin1_ref, blin1_ref, wlin2_ref, blin2_ref, o_ref):
    out = out_ref[:, :D]
    seg = batch_ref[...]
    cols = lax.broadcasted_iota(jnp.int32, (N_PAD, B), 1)
    S = (seg == cols).astype(jnp.float32)

    q_star = jnp.zeros((B, 2 * D), jnp.float32)
    hl = jnp.zeros((B, D), jnp.float32)
    cl = jnp.zeros((B, D), jnp.float32)
    for _ in range(STEPS):
        g = (
            jnp.dot(q_star, wli_ref[...], preferred_element_type=jnp.float32)
            + bli_ref[...]
            + jnp.dot(hl, wlh_ref[...], preferred_element_type=jnp.float32)
            + blh_ref[...]
        )
        ig = jax.nn.sigmoid(g[:, :D])
        fg = jax.nn.sigmoid(g[:, D:2 * D])
        cg = jnp.tanh(g[:, 2 * D:3 * D])
        og = jax.nn.sigmoid(g[:, 3 * D:])
        cl = fg * cl + ig * cg
        hl = og * jnp.tanh(cl)
        q = hl
        qn = jnp.dot(S, q, preferred_element_type=jnp.float32, precision=lax.Precision.HIGHEST)
        e = jnp.sum(out * qn, axis=1, keepdims=True)
        emax = jnp.max(jnp.where(S > 0, e, -jnp.inf), axis=0, keepdims=True)
        emax_n = jnp.sum(S * emax, axis=1, keepdims=True)
        ee = jnp.exp(e - emax_n)
        denom = lax.dot_general(
            S, ee, (((0,), (0,)), ((), ())),
            preferred_element_type=jnp.float32)
        denom_n = jnp.dot(S, denom, preferred_element_type=jnp.float32, precision=lax.Precision.HIGHEST)
        a = ee / (denom_n + 1e-16)
        rvec = lax.dot_general(
            S, a * out, (((0,), (0,)), ((), ())),
            preferred_element_type=jnp.float32)
        q_star = jnp.concatenate([q, rvec], axis=1)
    res = (
        jnp.dot(
            jax.nn.relu(
                jnp.dot(q_star, wlin1_ref[...], preferred_element_type=jnp.float32)
                + blin1_ref[...]
            ),
            wlin2_ref[...],
            preferred_element_type=jnp.float32,
        )
        + blin2_ref[...]
    )
    o_ref[...] = res


def _s2s(out, batch, W_li, W_lh, b_li, b_lh, W_lin1, b_lin1, W_lin2, b_lin2):
    return pl.pallas_call(
        _s2s_body,
        out_shape=jax.ShapeDtypeStruct((B, 1), jnp.float32),
    )(out, batch.reshape(N_PAD, 1), W_li.T, W_lh.T,
      b_li.reshape(1, 4 * D), b_lh.reshape(1, 4 * D),
      W_lin1, b_lin1.reshape(1, D), W_lin2, b_lin2.reshape(1, 1))


def kernel(x, edge_index, edge_attr, batch, W_lin0, b_lin0, W_e1, b_e1,
           W_e2, b_e2, W_root, b_conv, W_ih, W_hh, b_ih, b_hh,
           W_li, W_lh, b_li, b_lh, W_lin1, b_lin1, W_lin2, b_lin2):
    src = edge_index[0]
    dst = edge_index[1]

    out = _prep(x, W_lin0, b_lin0)

    W_ihT = W_ih.T
    W_hhT = W_hh.T

    zc = jnp.zeros((N_PAD, D), jnp.float32)
    zc16 = jnp.zeros((N_PAD, 16), jnp.float32)
    degp = jnp.stack([jax.ops.segment_sum(jnp.ones((E, 16), jnp.float32), dst,
                                          num_segments=N_PAD), zc16])

    for _ in range(JUMPS):
        x_j = jnp.take(out, src, axis=0)
        msg = _msg(edge_attr, x_j, W_e1, b_e1, W_e2, b_e2)
        p = jnp.stack([jax.ops.segment_sum(msg, dst, num_segments=N_PAD), zc])
        out = _gru(p, degp, out, W_root, b_conv, W_ihT, W_hhT, b_ih, b_hh)

    batch_pad = jnp.pad(batch, (0, N_PAD - N), constant_values=B)
    res = _s2s(out, batch_pad, W_li, W_lh, b_li, b_lh, W_lin1, b_lin1, W_lin2, b_lin2)
    return res.reshape(-1)
